# baseline (device time: 107361 ns/iter reference)
import jax
import jax.numpy as jnp
from jax import lax
from jax.experimental import pallas as pl
from jax.experimental.pallas import tpu as pltpu

N_DEV = 4
SQ = 1024
SKV = 1024
D_MODEL = 1024
HQ_PER = 8
DH = 128
HD_PER = HQ_PER * DH
SCALE = 0.08838834764831843
NEG_INF = -1e9


def kernel(x, Wq, K_ext, V_ext, Wo):
    my = lax.axis_index("i")
    xb = x[0].astype(jnp.bfloat16)
    wq_s = lax.dynamic_slice(Wq, (0, my * HD_PER), (D_MODEL, HD_PER)).astype(
        jnp.bfloat16
    )
    wo_s = lax.dynamic_slice(Wo, (my * HD_PER, 0), (HD_PER, D_MODEL)).astype(
        jnp.bfloat16
    )
    k_s = K_ext[0].transpose(1, 0, 2).astype(jnp.bfloat16)
    v_s = V_ext[0].transpose(1, 0, 2).astype(jnp.bfloat16)

    def body(x_ref, wq_ref, k_ref, v_ref, wo_ref, out_ref, ctx_ref, comm_ref,
             send_sems, recv_sems):
        my_pos = lax.axis_index("i")
        left = lax.rem(my_pos + (N_DEV - 1), N_DEV)
        right = lax.rem(my_pos + 1, N_DEV)

        q = jnp.dot(x_ref[:, :], wq_ref[:, :],
                    preferred_element_type=jnp.float32)
        qb = q.astype(jnp.bfloat16)

        qi = lax.broadcasted_iota(jnp.int32, (SQ, SKV), 0)
        ki = lax.broadcasted_iota(jnp.int32, (SQ, SKV), 1)
        mask = (jnp.abs(qi - ki) <= 128) | (ki < 32) | (qi < 32)

        for h in range(HQ_PER):
            qh = qb[:, h * DH:(h + 1) * DH]
            s = lax.dot_general(
                qh, k_ref[h], (((1,), (1,)), ((), ())),
                preferred_element_type=jnp.float32,
            ) * SCALE
            s = jnp.where(mask, s, NEG_INF)
            m = jnp.max(s, axis=1, keepdims=True)
            w = jnp.exp(s - m)
            w = w / jnp.sum(w, axis=1, keepdims=True)
            ctx_h = jnp.dot(w.astype(jnp.bfloat16), v_ref[h],
                            preferred_element_type=jnp.float32)
            ctx_ref[:, h * DH:(h + 1) * DH] = ctx_h.astype(jnp.bfloat16)

        partial = jnp.dot(ctx_ref[:, :], wo_ref[:, :],
                          preferred_element_type=jnp.float32)
        out_ref[0] = partial
        comm_ref[0] = partial.astype(jnp.bfloat16)

        barrier_sem = pltpu.get_barrier_semaphore()
        pl.semaphore_signal(barrier_sem, inc=1, device_id=(left,),
                            device_id_type=pl.DeviceIdType.MESH)
        pl.semaphore_signal(barrier_sem, inc=1, device_id=(right,),
                            device_id_type=pl.DeviceIdType.MESH)
        pl.semaphore_wait(barrier_sem, 2)

        for h in range(N_DEV - 1):
            rdma = pltpu.make_async_remote_copy(
                src_ref=comm_ref.at[h],
                dst_ref=comm_ref.at[h + 1],
                send_sem=send_sems.at[h],
                recv_sem=recv_sems.at[h],
                device_id=(right,),
                device_id_type=pl.DeviceIdType.MESH,
            )
            rdma.start()
            rdma.wait()
            out_ref[0] += comm_ref[h + 1].astype(jnp.float32)

    return pl.pallas_call(
        body,
        out_shape=jax.ShapeDtypeStruct((1, SQ, D_MODEL), jnp.float32),
        in_specs=[pl.BlockSpec(memory_space=pltpu.VMEM)] * 5,
        out_specs=pl.BlockSpec(memory_space=pltpu.VMEM),
        scratch_shapes=[
            pltpu.VMEM((SQ, HD_PER), jnp.bfloat16),
            pltpu.VMEM((N_DEV, SQ, D_MODEL), jnp.bfloat16),
            pltpu.SemaphoreType.DMA((N_DEV - 1,)),
            pltpu.SemaphoreType.DMA((N_DEV - 1,)),
        ],
        compiler_params=pltpu.CompilerParams(collective_id=0),
    )(xb, wq_s, k_s, v_s, wo_s)


# device time: 57833 ns/iter; 1.8564x vs baseline; 1.8564x over previous
import jax
import jax.numpy as jnp
from jax import lax
from jax.experimental import pallas as pl
from jax.experimental.pallas import tpu as pltpu

N_DEV = 4
SQ = 1024
SKV = 1024
D_MODEL = 1024
HQ_PER = 8
DH = 128
HD_PER = HQ_PER * DH
CHUNK = SQ // N_DEV
SCALE = 0.08838834764831843
NEG_INF = -1e9


def kernel(x, Wq, K_ext, V_ext, Wo):
    my = lax.axis_index("i")
    xb = x[0].astype(jnp.bfloat16)
    wq_s = lax.dynamic_slice(Wq, (0, my * HD_PER), (D_MODEL, HD_PER)).astype(
        jnp.bfloat16
    )
    wo_s = lax.dynamic_slice(Wo, (my * HD_PER, 0), (HD_PER, D_MODEL)).astype(
        jnp.bfloat16
    )
    k_s = K_ext[0].transpose(1, 0, 2).astype(jnp.bfloat16)
    v_s = V_ext[0].transpose(1, 0, 2).astype(jnp.bfloat16)

    def body(x_ref, wq_ref, k_ref, v_ref, wo_ref, out_ref,
             qbuf, cbuf, pbuf, sbufA, rbufA, sbufB, rbufB,
             sendA, recvA, sendB, recvB):
        my_pos = lax.axis_index("i")

        barrier_sem = pltpu.get_barrier_semaphore()
        for j in range(N_DEV - 1):
            peer = lax.rem(my_pos + 1 + j, N_DEV)
            pl.semaphore_signal(barrier_sem, inc=1, device_id=(peer,),
                                device_id_type=pl.DeviceIdType.MESH)
        pl.semaphore_wait(barrier_sem, N_DEV - 1)

        qbuf[:, :] = jnp.dot(
            x_ref[:, :], wq_ref[:, :], preferred_element_type=jnp.float32
        ).astype(jnp.bfloat16)

        ki = lax.broadcasted_iota(jnp.int32, (CHUNK, SKV), 1)

        def compute_chunk(off):
            qi = lax.broadcasted_iota(jnp.int32, (CHUNK, SKV), 0) + off
            mask = (jnp.abs(qi - ki) <= 128) | (ki < 32) | (qi < 32)
            for h in range(HQ_PER):
                qh = qbuf[pl.ds(off, CHUNK), h * DH:(h + 1) * DH]
                s = lax.dot_general(
                    qh, k_ref[h], (((1,), (1,)), ((), ())),
                    preferred_element_type=jnp.float32,
                ) * SCALE
                s = jnp.where(mask, s, NEG_INF)
                m = jnp.max(s, axis=1, keepdims=True)
                w = jnp.exp(s - m)
                w = w / jnp.sum(w, axis=1, keepdims=True)
                ctx_h = jnp.dot(w.astype(jnp.bfloat16), v_ref[h],
                                preferred_element_type=jnp.float32)
                cbuf[:, h * DH:(h + 1) * DH] = ctx_h.astype(jnp.bfloat16)
            return jnp.dot(cbuf[:, :], wo_ref[:, :],
                           preferred_element_type=jnp.float32)

        sendsA = []
        for j in range(N_DEV - 1):
            target = lax.rem(my_pos + 1 + j, N_DEV)
            sbufA[j] = compute_chunk(target * CHUNK).astype(jnp.bfloat16)
            rdma = pltpu.make_async_remote_copy(
                src_ref=sbufA.at[j],
                dst_ref=rbufA.at[2 - j],
                send_sem=sendA.at[j],
                recv_sem=recvA.at[2 - j],
                device_id=(target,),
                device_id_type=pl.DeviceIdType.MESH,
            )
            rdma.start()
            sendsA.append(rdma)

        pbuf[:, :] = compute_chunk(my_pos * CHUNK)

        for r in (2, 1, 0):
            recv = pltpu.make_async_remote_copy(
                src_ref=sbufA.at[0],
                dst_ref=rbufA.at[r],
                send_sem=sendA.at[0],
                recv_sem=recvA.at[r],
                device_id=(my_pos,),
                device_id_type=pl.DeviceIdType.MESH,
            )
            recv.wait_recv()
            pbuf[:, :] += rbufA[r].astype(jnp.float32)

        out_ref[0, pl.ds(my_pos * CHUNK, CHUNK), :] = pbuf[:, :]

        sbufB[:, :] = pbuf[:, :].astype(jnp.bfloat16)
        sendsB = []
        for j in range(N_DEV - 1):
            target = lax.rem(my_pos + 1 + j, N_DEV)
            rdma = pltpu.make_async_remote_copy(
                src_ref=sbufB,
                dst_ref=rbufB.at[2 - j],
                send_sem=sendB.at[j],
                recv_sem=recvB.at[2 - j],
                device_id=(target,),
                device_id_type=pl.DeviceIdType.MESH,
            )
            rdma.start()
            sendsB.append(rdma)

        for r in (2, 1, 0):
            recv = pltpu.make_async_remote_copy(
                src_ref=sbufB,
                dst_ref=rbufB.at[r],
                send_sem=sendB.at[0],
                recv_sem=recvB.at[r],
                device_id=(my_pos,),
                device_id_type=pl.DeviceIdType.MESH,
            )
            recv.wait_recv()
            src = lax.rem(my_pos + 1 + r, N_DEV)
            out_ref[0, pl.ds(src * CHUNK, CHUNK), :] = (
                rbufB[r].astype(jnp.float32)
            )

        for rdma in sendsA + sendsB:
            rdma.wait_send()

    return pl.pallas_call(
        body,
        out_shape=jax.ShapeDtypeStruct((1, SQ, D_MODEL), jnp.float32),
        in_specs=[pl.BlockSpec(memory_space=pltpu.VMEM)] * 5,
        out_specs=pl.BlockSpec(memory_space=pltpu.VMEM),
        scratch_shapes=[
            pltpu.VMEM((SQ, HD_PER), jnp.bfloat16),
            pltpu.VMEM((CHUNK, HD_PER), jnp.bfloat16),
            pltpu.VMEM((CHUNK, D_MODEL), jnp.float32),
            pltpu.VMEM((N_DEV - 1, CHUNK, D_MODEL), jnp.bfloat16),
            pltpu.VMEM((N_DEV - 1, CHUNK, D_MODEL), jnp.bfloat16),
            pltpu.VMEM((CHUNK, D_MODEL), jnp.bfloat16),
            pltpu.VMEM((N_DEV - 1, CHUNK, D_MODEL), jnp.bfloat16),
            pltpu.SemaphoreType.DMA((N_DEV - 1,)),
            pltpu.SemaphoreType.DMA((N_DEV - 1,)),
            pltpu.SemaphoreType.DMA((N_DEV - 1,)),
            pltpu.SemaphoreType.DMA((N_DEV - 1,)),
        ],
        compiler_params=pltpu.CompilerParams(collective_id=0),
    )(xb, wq_s, k_s, v_s, wo_s)


# device time: 57086 ns/iter; 1.8807x vs baseline; 1.0131x over previous
import jax
import jax.numpy as jnp
from jax import lax
from jax.experimental import pallas as pl
from jax.experimental.pallas import tpu as pltpu

N_DEV = 4
SQ = 1024
SKV = 1024
D_MODEL = 1024
HQ_PER = 8
DH = 128
HD_PER = HQ_PER * DH
CHUNK = SQ // N_DEV
SCALE = 0.08838834764831843
NEG_INF = -1e9


def kernel(x, Wq, K_ext, V_ext, Wo):
    my = lax.axis_index("i")
    xb = x[0].astype(jnp.bfloat16)
    wq_s = lax.dynamic_slice(Wq, (0, my * HD_PER), (D_MODEL, HD_PER)).astype(
        jnp.bfloat16
    )
    wo_s = lax.dynamic_slice(Wo, (my * HD_PER, 0), (HD_PER, D_MODEL)).astype(
        jnp.bfloat16
    )
    k_s = K_ext[0].transpose(1, 0, 2).astype(jnp.bfloat16)
    v_s = V_ext[0].transpose(1, 0, 2).astype(jnp.bfloat16)

    def body(x_ref, wq_ref, k_ref, v_ref, wo_ref, out_ref,
             qbuf, cbuf, pbuf, sbufA, rbufA, sbufB, rbufB,
             sendA, recvA, sendB, recvB):
        my_pos = lax.axis_index("i")

        barrier_sem = pltpu.get_barrier_semaphore()
        for j in range(N_DEV - 1):
            peer = lax.rem(my_pos + 1 + j, N_DEV)
            pl.semaphore_signal(barrier_sem, inc=1, device_id=(peer,),
                                device_id_type=pl.DeviceIdType.MESH)
        pl.semaphore_wait(barrier_sem, N_DEV - 1)

        qbuf[:, :] = jnp.dot(
            x_ref[:, :], wq_ref[:, :], preferred_element_type=jnp.float32
        ).astype(jnp.bfloat16)

        ki = lax.broadcasted_iota(jnp.int32, (CHUNK, SKV), 1)

        def compute_chunk(off):
            qi = lax.broadcasted_iota(jnp.int32, (CHUNK, SKV), 0) + off
            mask = (jnp.abs(qi - ki) <= 128) | (ki < 32) | (qi < 32)
            for h in range(HQ_PER):
                qh = qbuf[pl.ds(off, CHUNK), h * DH:(h + 1) * DH]
                s = lax.dot_general(
                    qh, k_ref[h], (((1,), (1,)), ((), ())),
                    preferred_element_type=jnp.float32,
                ) * SCALE
                s = jnp.where(mask, s, NEG_INF)
                m = jnp.max(s, axis=1, keepdims=True)
                w = jnp.exp(s - m)
                w = w / jnp.sum(w, axis=1, keepdims=True)
                ctx_h = jnp.dot(w.astype(jnp.bfloat16), v_ref[h],
                                preferred_element_type=jnp.float32)
                cbuf[:, h * DH:(h + 1) * DH] = ctx_h.astype(jnp.bfloat16)
            return jnp.dot(cbuf[:, :], wo_ref[:, :],
                           preferred_element_type=jnp.float32)

        sendsA = []
        for j in range(N_DEV - 1):
            target = lax.rem(my_pos + 1 + j, N_DEV)
            sbufA[j] = compute_chunk(target * CHUNK).astype(jnp.bfloat16)
            rdma = pltpu.make_async_remote_copy(
                src_ref=sbufA.at[j],
                dst_ref=rbufA.at[2 - j],
                send_sem=sendA.at[j],
                recv_sem=recvA.at[2 - j],
                device_id=(target,),
                device_id_type=pl.DeviceIdType.MESH,
            )
            rdma.start()
            sendsA.append(rdma)

        pbuf[:, :] = compute_chunk(my_pos * CHUNK)

        for r in (2, 1, 0):
            recv = pltpu.make_async_remote_copy(
                src_ref=sbufA.at[0],
                dst_ref=rbufA.at[r],
                send_sem=sendA.at[0],
                recv_sem=recvA.at[r],
                device_id=(my_pos,),
                device_id_type=pl.DeviceIdType.MESH,
            )
            recv.wait_recv()
            pbuf[:, :] += rbufA[r].astype(jnp.float32)

        sbufB[:, :] = pbuf[:, :].astype(jnp.bfloat16)
        out_ref[0, pl.ds(my_pos * CHUNK, CHUNK), :] = sbufB[:, :]
        sendsB = []
        for j in range(N_DEV - 1):
            target = lax.rem(my_pos + 1 + j, N_DEV)
            rdma = pltpu.make_async_remote_copy(
                src_ref=sbufB,
                dst_ref=rbufB.at[2 - j],
                send_sem=sendB.at[j],
                recv_sem=recvB.at[2 - j],
                device_id=(target,),
                device_id_type=pl.DeviceIdType.MESH,
            )
            rdma.start()
            sendsB.append(rdma)

        for r in (2, 1, 0):
            recv = pltpu.make_async_remote_copy(
                src_ref=sbufB,
                dst_ref=rbufB.at[r],
                send_sem=sendB.at[0],
                recv_sem=recvB.at[r],
                device_id=(my_pos,),
                device_id_type=pl.DeviceIdType.MESH,
            )
            recv.wait_recv()
            src = lax.rem(my_pos + 1 + r, N_DEV)
            out_ref[0, pl.ds(src * CHUNK, CHUNK), :] = rbufB[r]

        for rdma in sendsA + sendsB:
            rdma.wait_send()

    return pl.pallas_call(
        body,
        out_shape=jax.ShapeDtypeStruct((1, SQ, D_MODEL), jnp.bfloat16),
        in_specs=[pl.BlockSpec(memory_space=pltpu.VMEM)] * 5,
        out_specs=pl.BlockSpec(memory_space=pltpu.VMEM),
        scratch_shapes=[
            pltpu.VMEM((SQ, HD_PER), jnp.bfloat16),
            pltpu.VMEM((CHUNK, HD_PER), jnp.bfloat16),
            pltpu.VMEM((CHUNK, D_MODEL), jnp.float32),
            pltpu.VMEM((N_DEV - 1, CHUNK, D_MODEL), jnp.bfloat16),
            pltpu.VMEM((N_DEV - 1, CHUNK, D_MODEL), jnp.bfloat16),
            pltpu.VMEM((CHUNK, D_MODEL), jnp.bfloat16),
            pltpu.VMEM((N_DEV - 1, CHUNK, D_MODEL), jnp.bfloat16),
            pltpu.SemaphoreType.DMA((N_DEV - 1,)),
            pltpu.SemaphoreType.DMA((N_DEV - 1,)),
            pltpu.SemaphoreType.DMA((N_DEV - 1,)),
            pltpu.SemaphoreType.DMA((N_DEV - 1,)),
        ],
        compiler_params=pltpu.CompilerParams(collective_id=0),
    )(xb, wq_s, k_s, v_s, wo_s)


# device time: 54134 ns/iter; 1.9832x vs baseline; 1.0545x over previous
import jax
import jax.numpy as jnp
from jax import lax
from jax.experimental import pallas as pl
from jax.experimental.pallas import tpu as pltpu

N_DEV = 4
SQ = 1024
SKV = 1024
D_MODEL = 1024
HQ_PER = 8
DH = 128
HD_PER = HQ_PER * DH
CHUNK = SQ // N_DEV
SCALE = 0.08838834764831843
NEG_INF = -1e9


def kernel(x, Wq, K_ext, V_ext, Wo):
    def body(x_hbm, wq_hbm, k_hbm, v_hbm, wo_hbm, out_ref,
             xf, wqf, wof, kf, vf, xb, wqb, wob, kb, vb,
             qbuf, cbuf, pbuf, sbufA, rbufA, sbufB, rbufB,
             copy_sems, sendA, recvA, sendB, recvB):
        my_pos = lax.axis_index("i")

        dmas = []
        for src, dst, i in (
            (x_hbm.at[0], xf, 0),
            (wq_hbm.at[:, pl.ds(my_pos * HD_PER, HD_PER)], wqf, 1),
            (wo_hbm.at[pl.ds(my_pos * HD_PER, HD_PER), :], wof, 2),
        ):
            c = pltpu.make_async_copy(src, dst, copy_sems.at[i])
            c.start()
            dmas.append(c)
        kv_dmas = []
        for h in range(HQ_PER):
            ck = pltpu.make_async_copy(
                k_hbm.at[0, :, h, :], kf.at[h], copy_sems.at[3]
            )
            cv = pltpu.make_async_copy(
                v_hbm.at[0, :, h, :], vf.at[h], copy_sems.at[4]
            )
            ck.start()
            cv.start()
            kv_dmas.append((ck, cv))

        barrier_sem = pltpu.get_barrier_semaphore()
        for j in range(N_DEV - 1):
            peer = lax.rem(my_pos + 1 + j, N_DEV)
            pl.semaphore_signal(barrier_sem, inc=1, device_id=(peer,),
                                device_id_type=pl.DeviceIdType.MESH)
        pl.semaphore_wait(barrier_sem, N_DEV - 1)

        dmas[0].wait()
        dmas[1].wait()
        xb[:, :] = xf[:, :].astype(jnp.bfloat16)
        wqb[:, :] = wqf[:, :].astype(jnp.bfloat16)
        qbuf[:, :] = jnp.dot(
            xb[:, :], wqb[:, :], preferred_element_type=jnp.float32
        ).astype(jnp.bfloat16)

        for ck, cv in kv_dmas:
            ck.wait()
            cv.wait()
        for h in range(HQ_PER):
            kb[h] = kf[h].astype(jnp.bfloat16)
            vb[h] = vf[h].astype(jnp.bfloat16)
        dmas[2].wait()
        wob[:, :] = wof[:, :].astype(jnp.bfloat16)

        ki = lax.broadcasted_iota(jnp.int32, (CHUNK, SKV), 1)

        def compute_chunk(off):
            qi = lax.broadcasted_iota(jnp.int32, (CHUNK, SKV), 0) + off
            mask = (jnp.abs(qi - ki) <= 128) | (ki < 32) | (qi < 32)
            for h in range(HQ_PER):
                qh = qbuf[pl.ds(off, CHUNK), h * DH:(h + 1) * DH]
                s = lax.dot_general(
                    qh, kb[h], (((1,), (1,)), ((), ())),
                    preferred_element_type=jnp.float32,
                ) * SCALE
                s = jnp.where(mask, s, NEG_INF)
                m = jnp.max(s, axis=1, keepdims=True)
                w = jnp.exp(s - m)
                w = w / jnp.sum(w, axis=1, keepdims=True)
                ctx_h = jnp.dot(w.astype(jnp.bfloat16), vb[h],
                                preferred_element_type=jnp.float32)
                cbuf[:, h * DH:(h + 1) * DH] = ctx_h.astype(jnp.bfloat16)
            return jnp.dot(cbuf[:, :], wob[:, :],
                           preferred_element_type=jnp.float32)

        sendsA = []
        for j in range(N_DEV - 1):
            target = lax.rem(my_pos + 1 + j, N_DEV)
            sbufA[j] = compute_chunk(target * CHUNK).astype(jnp.bfloat16)
            rdma = pltpu.make_async_remote_copy(
                src_ref=sbufA.at[j],
                dst_ref=rbufA.at[2 - j],
                send_sem=sendA.at[j],
                recv_sem=recvA.at[2 - j],
                device_id=(target,),
                device_id_type=pl.DeviceIdType.MESH,
            )
            rdma.start()
            sendsA.append(rdma)

        pbuf[:, :] = compute_chunk(my_pos * CHUNK)

        for r in (2, 1, 0):
            recv = pltpu.make_async_remote_copy(
                src_ref=sbufA.at[0],
                dst_ref=rbufA.at[r],
                send_sem=sendA.at[0],
                recv_sem=recvA.at[r],
                device_id=(my_pos,),
                device_id_type=pl.DeviceIdType.MESH,
            )
            recv.wait_recv()
            pbuf[:, :] += rbufA[r].astype(jnp.float32)

        sbufB[:, :] = pbuf[:, :].astype(jnp.bfloat16)
        out_ref[0, pl.ds(my_pos * CHUNK, CHUNK), :] = sbufB[:, :]
        sendsB = []
        for j in range(N_DEV - 1):
            target = lax.rem(my_pos + 1 + j, N_DEV)
            rdma = pltpu.make_async_remote_copy(
                src_ref=sbufB,
                dst_ref=rbufB.at[2 - j],
                send_sem=sendB.at[j],
                recv_sem=recvB.at[2 - j],
                device_id=(target,),
                device_id_type=pl.DeviceIdType.MESH,
            )
            rdma.start()
            sendsB.append(rdma)

        for r in (2, 1, 0):
            recv = pltpu.make_async_remote_copy(
                src_ref=sbufB,
                dst_ref=rbufB.at[r],
                send_sem=sendB.at[0],
                recv_sem=recvB.at[r],
                device_id=(my_pos,),
                device_id_type=pl.DeviceIdType.MESH,
            )
            recv.wait_recv()
            src = lax.rem(my_pos + 1 + r, N_DEV)
            out_ref[0, pl.ds(src * CHUNK, CHUNK), :] = rbufB[r]

        for rdma in sendsA + sendsB:
            rdma.wait_send()

    return pl.pallas_call(
        body,
        out_shape=jax.ShapeDtypeStruct((1, SQ, D_MODEL), jnp.bfloat16),
        in_specs=[pl.BlockSpec(memory_space=pl.ANY)] * 5,
        out_specs=pl.BlockSpec(memory_space=pltpu.VMEM),
        scratch_shapes=[
            pltpu.VMEM((SQ, D_MODEL), jnp.float32),
            pltpu.VMEM((D_MODEL, HD_PER), jnp.float32),
            pltpu.VMEM((HD_PER, D_MODEL), jnp.float32),
            pltpu.VMEM((HQ_PER, SKV, DH), jnp.float32),
            pltpu.VMEM((HQ_PER, SKV, DH), jnp.float32),
            pltpu.VMEM((SQ, D_MODEL), jnp.bfloat16),
            pltpu.VMEM((D_MODEL, HD_PER), jnp.bfloat16),
            pltpu.VMEM((HD_PER, D_MODEL), jnp.bfloat16),
            pltpu.VMEM((HQ_PER, SKV, DH), jnp.bfloat16),
            pltpu.VMEM((HQ_PER, SKV, DH), jnp.bfloat16),
            pltpu.VMEM((SQ, HD_PER), jnp.bfloat16),
            pltpu.VMEM((CHUNK, HD_PER), jnp.bfloat16),
            pltpu.VMEM((CHUNK, D_MODEL), jnp.float32),
            pltpu.VMEM((N_DEV - 1, CHUNK, D_MODEL), jnp.bfloat16),
            pltpu.VMEM((N_DEV - 1, CHUNK, D_MODEL), jnp.bfloat16),
            pltpu.VMEM((CHUNK, D_MODEL), jnp.bfloat16),
            pltpu.VMEM((N_DEV - 1, CHUNK, D_MODEL), jnp.bfloat16),
            pltpu.SemaphoreType.DMA((5,)),
            pltpu.SemaphoreType.DMA((N_DEV - 1,)),
            pltpu.SemaphoreType.DMA((N_DEV - 1,)),
            pltpu.SemaphoreType.DMA((N_DEV - 1,)),
            pltpu.SemaphoreType.DMA((N_DEV - 1,)),
        ],
        compiler_params=pltpu.CompilerParams(
            collective_id=0, vmem_limit_bytes=64 * 1024 * 1024
        ),
    )(x, Wq, K_ext, V_ext, Wo)


# device time: 51463 ns/iter; 2.0862x vs baseline; 1.0519x over previous
import jax
import jax.numpy as jnp
from jax import lax
from jax.experimental import pallas as pl
from jax.experimental.pallas import tpu as pltpu

N_DEV = 4
SQ = 1024
SKV = 1024
D_MODEL = 1024
HQ_PER = 8
DH = 128
HD_PER = HQ_PER * DH
CHUNK = SQ // N_DEV
SCALE = 0.08838834764831843
NEG_INF = -1e9


def kernel(x, Wq, K_ext, V_ext, Wo):
    def body(x_hbm, wq_hbm, k_hbm, v_hbm, wo_hbm, out_ref,
             xf, wqf, wof, kf, vf, xb, wqb, wob, kb, vb,
             qbuf, cbuf, pbuf, sbufA, rbufA, sbufB, rbufB,
             copy_sems, sendA, recvA, sendB, recvB):
        my_pos = lax.axis_index("i")

        dmas = []
        for src, dst, i in (
            (x_hbm.at[0], xf, 0),
            (wq_hbm.at[:, pl.ds(my_pos * HD_PER, HD_PER)], wqf, 1),
            (wo_hbm.at[pl.ds(my_pos * HD_PER, HD_PER), :], wof, 2),
        ):
            c = pltpu.make_async_copy(src, dst, copy_sems.at[i])
            c.start()
            dmas.append(c)
        kv_dmas = []
        for h in range(HQ_PER):
            ck = pltpu.make_async_copy(
                k_hbm.at[0, :, h, :], kf.at[h], copy_sems.at[3]
            )
            cv = pltpu.make_async_copy(
                v_hbm.at[0, :, h, :], vf.at[h], copy_sems.at[4]
            )
            ck.start()
            cv.start()
            kv_dmas.append((ck, cv))

        barrier_sem = pltpu.get_barrier_semaphore()
        for j in range(N_DEV - 1):
            peer = lax.rem(my_pos + 1 + j, N_DEV)
            pl.semaphore_signal(barrier_sem, inc=1, device_id=(peer,),
                                device_id_type=pl.DeviceIdType.MESH)
        pl.semaphore_wait(barrier_sem, N_DEV - 1)

        dmas[0].wait()
        dmas[1].wait()
        xb[:, :] = xf[:, :].astype(jnp.bfloat16)
        wqb[:, :] = wqf[:, :].astype(jnp.bfloat16)
        qbuf[:, :] = jnp.dot(
            xb[:, :], wqb[:, :], preferred_element_type=jnp.float32
        ).astype(jnp.bfloat16)

        for ck, cv in kv_dmas:
            ck.wait()
            cv.wait()
        for h in range(HQ_PER):
            kb[h] = kf[h].astype(jnp.bfloat16)
            vb[h] = vf[h].astype(jnp.bfloat16)
        dmas[2].wait()
        wob[:, :] = wof[:, :].astype(jnp.bfloat16)

        ki = lax.broadcasted_iota(jnp.int32, (CHUNK, SKV), 1)

        def compute_chunk(off):
            qi = lax.broadcasted_iota(jnp.int32, (CHUNK, SKV), 0) + off
            mask = (jnp.abs(qi - ki) <= 128) | (ki < 32) | (qi < 32)
            for h in range(HQ_PER):
                qh = qbuf[pl.ds(off, CHUNK), h * DH:(h + 1) * DH]
                s = lax.dot_general(
                    qh, kb[h], (((1,), (1,)), ((), ())),
                    preferred_element_type=jnp.float32,
                ) * SCALE
                s = jnp.where(mask, s, NEG_INF)
                w = jnp.exp(s)
                w = w * (1.0 / jnp.sum(w, axis=1, keepdims=True))
                ctx_h = jnp.dot(w.astype(jnp.bfloat16), vb[h],
                                preferred_element_type=jnp.float32)
                cbuf[:, h * DH:(h + 1) * DH] = ctx_h.astype(jnp.bfloat16)
            return jnp.dot(cbuf[:, :], wob[:, :],
                           preferred_element_type=jnp.float32)

        sendsA = []
        for j in range(N_DEV - 1):
            target = lax.rem(my_pos + 1 + j, N_DEV)
            sbufA[j] = compute_chunk(target * CHUNK).astype(jnp.bfloat16)
            rdma = pltpu.make_async_remote_copy(
                src_ref=sbufA.at[j],
                dst_ref=rbufA.at[2 - j],
                send_sem=sendA.at[j],
                recv_sem=recvA.at[2 - j],
                device_id=(target,),
                device_id_type=pl.DeviceIdType.MESH,
            )
            rdma.start()
            sendsA.append(rdma)

        pbuf[:, :] = compute_chunk(my_pos * CHUNK)

        for r in (2, 1, 0):
            recv = pltpu.make_async_remote_copy(
                src_ref=sbufA.at[0],
                dst_ref=rbufA.at[r],
                send_sem=sendA.at[0],
                recv_sem=recvA.at[r],
                device_id=(my_pos,),
                device_id_type=pl.DeviceIdType.MESH,
            )
            recv.wait_recv()
            pbuf[:, :] += rbufA[r].astype(jnp.float32)

        sbufB[:, :] = pbuf[:, :].astype(jnp.bfloat16)
        out_ref[0, pl.ds(my_pos * CHUNK, CHUNK), :] = sbufB[:, :]
        sendsB = []
        for j in range(N_DEV - 1):
            target = lax.rem(my_pos + 1 + j, N_DEV)
            rdma = pltpu.make_async_remote_copy(
                src_ref=sbufB,
                dst_ref=rbufB.at[2 - j],
                send_sem=sendB.at[j],
                recv_sem=recvB.at[2 - j],
                device_id=(target,),
                device_id_type=pl.DeviceIdType.MESH,
            )
            rdma.start()
            sendsB.append(rdma)

        for r in (2, 1, 0):
            recv = pltpu.make_async_remote_copy(
                src_ref=sbufB,
                dst_ref=rbufB.at[r],
                send_sem=sendB.at[0],
                recv_sem=recvB.at[r],
                device_id=(my_pos,),
                device_id_type=pl.DeviceIdType.MESH,
            )
            recv.wait_recv()
            src = lax.rem(my_pos + 1 + r, N_DEV)
            out_ref[0, pl.ds(src * CHUNK, CHUNK), :] = rbufB[r]

        for rdma in sendsA + sendsB:
            rdma.wait_send()

    return pl.pallas_call(
        body,
        out_shape=jax.ShapeDtypeStruct((1, SQ, D_MODEL), jnp.bfloat16),
        in_specs=[pl.BlockSpec(memory_space=pl.ANY)] * 5,
        out_specs=pl.BlockSpec(memory_space=pltpu.VMEM),
        scratch_shapes=[
            pltpu.VMEM((SQ, D_MODEL), jnp.float32),
            pltpu.VMEM((D_MODEL, HD_PER), jnp.float32),
            pltpu.VMEM((HD_PER, D_MODEL), jnp.float32),
            pltpu.VMEM((HQ_PER, SKV, DH), jnp.float32),
            pltpu.VMEM((HQ_PER, SKV, DH), jnp.float32),
            pltpu.VMEM((SQ, D_MODEL), jnp.bfloat16),
            pltpu.VMEM((D_MODEL, HD_PER), jnp.bfloat16),
            pltpu.VMEM((HD_PER, D_MODEL), jnp.bfloat16),
            pltpu.VMEM((HQ_PER, SKV, DH), jnp.bfloat16),
            pltpu.VMEM((HQ_PER, SKV, DH), jnp.bfloat16),
            pltpu.VMEM((SQ, HD_PER), jnp.bfloat16),
            pltpu.VMEM((CHUNK, HD_PER), jnp.bfloat16),
            pltpu.VMEM((CHUNK, D_MODEL), jnp.float32),
            pltpu.VMEM((N_DEV - 1, CHUNK, D_MODEL), jnp.bfloat16),
            pltpu.VMEM((N_DEV - 1, CHUNK, D_MODEL), jnp.bfloat16),
            pltpu.VMEM((CHUNK, D_MODEL), jnp.bfloat16),
            pltpu.VMEM((N_DEV - 1, CHUNK, D_MODEL), jnp.bfloat16),
            pltpu.SemaphoreType.DMA((5,)),
            pltpu.SemaphoreType.DMA((N_DEV - 1,)),
            pltpu.SemaphoreType.DMA((N_DEV - 1,)),
            pltpu.SemaphoreType.DMA((N_DEV - 1,)),
            pltpu.SemaphoreType.DMA((N_DEV - 1,)),
        ],
        compiler_params=pltpu.CompilerParams(
            collective_id=0, vmem_limit_bytes=64 * 1024 * 1024
        ),
    )(x, Wq, K_ext, V_ext, Wo)


# device time: 50393 ns/iter; 2.1305x vs baseline; 1.0212x over previous
import jax
import jax.numpy as jnp
from jax import lax
from jax.experimental import pallas as pl
from jax.experimental.pallas import tpu as pltpu

N_DEV = 4
SQ = 1024
SKV = 1024
D_MODEL = 1024
HQ_PER = 8
DH = 128
HD_PER = HQ_PER * DH
CHUNK = SQ // N_DEV
SCALE = 0.08838834764831843
NEG_INF = -1e9


def kernel(x, Wq, K_ext, V_ext, Wo):
    def body(x_hbm, wq_hbm, k_hbm, v_hbm, wo_hbm, out_ref,
             xf, wqf, wof, kf, vf, xb, wqb, wob, kb, vb,
             qbuf, cbuf, pbuf, sbufA, rbufA, sbufB, rbufB,
             copy_sems, sendA, recvA, sendB, recvB):
        my_pos = lax.axis_index("i")

        dmas = []
        for src, dst, i in (
            (x_hbm.at[0], xf, 0),
            (wq_hbm.at[:, pl.ds(my_pos * HD_PER, HD_PER)], wqf, 1),
            (wo_hbm.at[pl.ds(my_pos * HD_PER, HD_PER), :], wof, 2),
        ):
            c = pltpu.make_async_copy(src, dst, copy_sems.at[i])
            c.start()
            dmas.append(c)
        kv_dmas = []
        for h in range(HQ_PER):
            ck = pltpu.make_async_copy(
                k_hbm.at[0, :, h, :], kf.at[h], copy_sems.at[3]
            )
            cv = pltpu.make_async_copy(
                v_hbm.at[0, :, h, :], vf.at[h], copy_sems.at[4]
            )
            ck.start()
            cv.start()
            kv_dmas.append((ck, cv))

        barrier_sem = pltpu.get_barrier_semaphore()
        for j in range(N_DEV - 1):
            peer = lax.rem(my_pos + 1 + j, N_DEV)
            pl.semaphore_signal(barrier_sem, inc=1, device_id=(peer,),
                                device_id_type=pl.DeviceIdType.MESH)
        pl.semaphore_wait(barrier_sem, N_DEV - 1)

        dmas[0].wait()
        dmas[1].wait()
        xb[:, :] = xf[:, :].astype(jnp.bfloat16)
        wqb[:, :] = wqf[:, :].astype(jnp.bfloat16)
        qbuf[:, :] = jnp.dot(
            xb[:, :], wqb[:, :], preferred_element_type=jnp.float32
        ).astype(jnp.bfloat16)

        for ck, cv in kv_dmas:
            ck.wait()
            cv.wait()
        for h in range(HQ_PER):
            kb[h] = kf[h].astype(jnp.bfloat16)
            vb[h] = vf[h].astype(jnp.bfloat16)
        dmas[2].wait()
        wob[:, :] = wof[:, :].astype(jnp.bfloat16)

        BAND = 512
        GLOB = 128

        def compute_chunk(off):

            @pl.when(off == 0)
            def _dense():
                qi = lax.broadcasted_iota(jnp.int32, (CHUNK, SKV), 0) + off
                ki = lax.broadcasted_iota(jnp.int32, (CHUNK, SKV), 1)
                mask = (jnp.abs(qi - ki) <= 128) | (ki < 32) | (qi < 32)
                for h in range(HQ_PER):
                    qh = qbuf[pl.ds(off, CHUNK), h * DH:(h + 1) * DH]
                    s = lax.dot_general(
                        qh, kb[h], (((1,), (1,)), ((), ())),
                        preferred_element_type=jnp.float32,
                    ) * SCALE
                    w = jnp.exp(jnp.where(mask, s, NEG_INF))
                    w = w * (1.0 / jnp.sum(w, axis=1, keepdims=True))
                    ctx_h = jnp.dot(w.astype(jnp.bfloat16), vb[h],
                                    preferred_element_type=jnp.float32)
                    cbuf[:, h * DH:(h + 1) * DH] = ctx_h.astype(jnp.bfloat16)

            @pl.when(off != 0)
            def _sparse():
                bs = jnp.minimum(off - 128, SKV - BAND)
                qi = lax.broadcasted_iota(jnp.int32, (CHUNK, BAND), 0) + off
                kib = lax.broadcasted_iota(jnp.int32, (CHUNK, BAND), 1) + bs
                mask_b = jnp.abs(qi - kib) <= 128
                kig = lax.broadcasted_iota(jnp.int32, (CHUNK, GLOB), 1)
                mask_g = kig < 32
                for h in range(HQ_PER):
                    qh = qbuf[pl.ds(off, CHUNK), h * DH:(h + 1) * DH]
                    s_b = lax.dot_general(
                        qh, kb[h, pl.ds(bs, BAND), :],
                        (((1,), (1,)), ((), ())),
                        preferred_element_type=jnp.float32,
                    ) * SCALE
                    s_g = lax.dot_general(
                        qh, kb[h, 0:GLOB, :], (((1,), (1,)), ((), ())),
                        preferred_element_type=jnp.float32,
                    ) * SCALE
                    wb = jnp.exp(jnp.where(mask_b, s_b, NEG_INF))
                    wg = jnp.exp(jnp.where(mask_g, s_g, NEG_INF))
                    r = 1.0 / (jnp.sum(wb, axis=1, keepdims=True)
                               + jnp.sum(wg, axis=1, keepdims=True))
                    ctx_h = jnp.dot(
                        (wb * r).astype(jnp.bfloat16), vb[h, pl.ds(bs, BAND), :],
                        preferred_element_type=jnp.float32,
                    ) + jnp.dot(
                        (wg * r).astype(jnp.bfloat16), vb[h, 0:GLOB, :],
                        preferred_element_type=jnp.float32,
                    )
                    cbuf[:, h * DH:(h + 1) * DH] = ctx_h.astype(jnp.bfloat16)

            return jnp.dot(cbuf[:, :], wob[:, :],
                           preferred_element_type=jnp.float32)

        sendsA = []
        for j in range(N_DEV - 1):
            target = lax.rem(my_pos + 1 + j, N_DEV)
            sbufA[j] = compute_chunk(target * CHUNK).astype(jnp.bfloat16)
            rdma = pltpu.make_async_remote_copy(
                src_ref=sbufA.at[j],
                dst_ref=rbufA.at[2 - j],
                send_sem=sendA.at[j],
                recv_sem=recvA.at[2 - j],
                device_id=(target,),
                device_id_type=pl.DeviceIdType.MESH,
            )
            rdma.start()
            sendsA.append(rdma)

        pbuf[:, :] = compute_chunk(my_pos * CHUNK)

        for r in (2, 1, 0):
            recv = pltpu.make_async_remote_copy(
                src_ref=sbufA.at[0],
                dst_ref=rbufA.at[r],
                send_sem=sendA.at[0],
                recv_sem=recvA.at[r],
                device_id=(my_pos,),
                device_id_type=pl.DeviceIdType.MESH,
            )
            recv.wait_recv()
            pbuf[:, :] += rbufA[r].astype(jnp.float32)

        sbufB[:, :] = pbuf[:, :].astype(jnp.bfloat16)
        out_ref[0, pl.ds(my_pos * CHUNK, CHUNK), :] = sbufB[:, :]
        sendsB = []
        for j in range(N_DEV - 1):
            target = lax.rem(my_pos + 1 + j, N_DEV)
            rdma = pltpu.make_async_remote_copy(
                src_ref=sbufB,
                dst_ref=rbufB.at[2 - j],
                send_sem=sendB.at[j],
                recv_sem=recvB.at[2 - j],
                device_id=(target,),
                device_id_type=pl.DeviceIdType.MESH,
            )
            rdma.start()
            sendsB.append(rdma)

        for r in (2, 1, 0):
            recv = pltpu.make_async_remote_copy(
                src_ref=sbufB,
                dst_ref=rbufB.at[r],
                send_sem=sendB.at[0],
                recv_sem=recvB.at[r],
                device_id=(my_pos,),
                device_id_type=pl.DeviceIdType.MESH,
            )
            recv.wait_recv()
            src = lax.rem(my_pos + 1 + r, N_DEV)
            out_ref[0, pl.ds(src * CHUNK, CHUNK), :] = rbufB[r]

        for rdma in sendsA + sendsB:
            rdma.wait_send()

    return pl.pallas_call(
        body,
        out_shape=jax.ShapeDtypeStruct((1, SQ, D_MODEL), jnp.bfloat16),
        in_specs=[pl.BlockSpec(memory_space=pl.ANY)] * 5,
        out_specs=pl.BlockSpec(memory_space=pltpu.VMEM),
        scratch_shapes=[
            pltpu.VMEM((SQ, D_MODEL), jnp.float32),
            pltpu.VMEM((D_MODEL, HD_PER), jnp.float32),
            pltpu.VMEM((HD_PER, D_MODEL), jnp.float32),
            pltpu.VMEM((HQ_PER, SKV, DH), jnp.float32),
            pltpu.VMEM((HQ_PER, SKV, DH), jnp.float32),
            pltpu.VMEM((SQ, D_MODEL), jnp.bfloat16),
            pltpu.VMEM((D_MODEL, HD_PER), jnp.bfloat16),
            pltpu.VMEM((HD_PER, D_MODEL), jnp.bfloat16),
            pltpu.VMEM((HQ_PER, SKV, DH), jnp.bfloat16),
            pltpu.VMEM((HQ_PER, SKV, DH), jnp.bfloat16),
            pltpu.VMEM((SQ, HD_PER), jnp.bfloat16),
            pltpu.VMEM((CHUNK, HD_PER), jnp.bfloat16),
            pltpu.VMEM((CHUNK, D_MODEL), jnp.float32),
            pltpu.VMEM((N_DEV - 1, CHUNK, D_MODEL), jnp.bfloat16),
            pltpu.VMEM((N_DEV - 1, CHUNK, D_MODEL), jnp.bfloat16),
            pltpu.VMEM((CHUNK, D_MODEL), jnp.bfloat16),
            pltpu.VMEM((N_DEV - 1, CHUNK, D_MODEL), jnp.bfloat16),
            pltpu.SemaphoreType.DMA((5,)),
            pltpu.SemaphoreType.DMA((N_DEV - 1,)),
            pltpu.SemaphoreType.DMA((N_DEV - 1,)),
            pltpu.SemaphoreType.DMA((N_DEV - 1,)),
            pltpu.SemaphoreType.DMA((N_DEV - 1,)),
        ],
        compiler_params=pltpu.CompilerParams(
            collective_id=0, vmem_limit_bytes=64 * 1024 * 1024
        ),
    )(x, Wq, K_ext, V_ext, Wo)
